# baseline (speedup 1.0000x reference)
import functools

import jax
import jax.numpy as jnp
from jax import lax
from jax.experimental import pallas as pl
from jax.experimental.pallas import tpu as pltpu
from jax.experimental.pallas import tpu_sc as plsc

_B = 4096
_F = 26
_D = 16
_N = _B * _F
_NW = 32
_CHUNK = _N // _NW
_V = 1000000
_TAU = 0.1
_SIG_HALF = 1.0 / (1.0 + 2.718281828459045 ** (-0.5))

_S = 65536
_C = _S // 8
_LC = _C.bit_length() - 1
_GRID = (_V + _S - 1) // _S
_VP = _GRID * _S


def _tr_body(wt_ref, sm_ref, out_ref):
    j = pl.program_id(0)
    col0 = j * _S
    sm = sm_ref[...]
    scale = jnp.float32(1.0 / _SIG_HALF) / (
        1.0 + jnp.exp(sm * jnp.float32(-1.0 / _TAU)))
    w = wt_ref[...] * scale
    gcol = col0 + lax.broadcasted_iota(jnp.int32, (1, _S), 1)
    w = jnp.where(gcol < _V, w, 0.0)
    lhs = jnp.concatenate([w[:, k * _C:(k + 1) * _C] for k in range(8)],
                          axis=0)
    out_ref[...] = lax.dot_general(
        lhs, jnp.eye(128, dtype=jnp.float32), (((0,), (0,)), ((), ())),
        preferred_element_type=jnp.float32)


def _permute_tc(wt, smt):
    out = pl.pallas_call(
        _tr_body,
        grid=(_GRID,),
        in_specs=[pl.BlockSpec((_D, _S), lambda j: (0, j)),
                  pl.BlockSpec((1, _S), lambda j: (0, j))],
        out_specs=pl.BlockSpec((_S // 8, 128), lambda j: (j, 0)),
        out_shape=jax.ShapeDtypeStruct((_VP * _D // 128, 128), jnp.float32),
    )(wt, smt)
    return out.reshape(_VP, _D)


_NPH = 4
_H = _CHUNK // _NPH
_LH = 128 // _NPH


def _sc_body(x_hbm, w_hbm, out_hbm, idx_v, idxp, rows, p5_v, sems):
    wid = lax.axis_index("s") * 2 + lax.axis_index("c")
    base = wid * _CHUNK
    pltpu.sync_copy(x_hbm.at[pl.ds(base, _CHUNK)], idx_v)

    def perm_phase(off, dst):
        @plsc.parallel_loop(0, _H // 16, unroll=4)
        def body(g):
            i = idx_v[pl.ds(off + g * 16, 16)]
            p = (i & ~(_S - 1)) | ((i & (_C - 1)) << 3) | ((i >> _LC) & 7)
            dst[pl.ds(g * 16, 16)] = p

    copies = []
    for ph in range(_NPH):
        perm_phase(ph * _H, idxp[ph])
        copies.append(pltpu.async_copy(w_hbm.at[idxp[ph]], rows[ph],
                                       sems[ph]))

    d_iota = lax.broadcasted_iota(jnp.int32, (16,), 0)
    ts_vec = d_iota >> 3
    s_vec = d_iota & 7
    zero_vec = jnp.zeros((16,), jnp.int32)
    one_vec = jnp.ones((16,), jnp.int32)

    def transpose_phase(rows_ref, l0):
        l0_vec = jnp.full((16,), l0, jnp.int32)

        def f_body(f, f_vec):
            @plsc.parallel_loop(0, _LH, unroll=8)
            def l_body(l):
                vals = rows_ref[l * _F + f, :]
                l_vec = l0_vec + l
                plsc.store_scatter(
                    p5_v, [f_vec, ts_vec, zero_vec, s_vec, l_vec], vals)

            return f_vec + one_vec

        lax.fori_loop(0, _F, f_body, zero_vec)

    for ph in range(_NPH):
        copies[ph].wait()
        transpose_phase(rows[ph], ph * _LH)
    pltpu.sync_copy(p5_v.at[:, :, :, :, pl.ds(0, 128)],
                    out_hbm.at[:, :, pl.ds(wid, 1)])


def _sc_lookup(x_flat, w_perm):
    mesh = plsc.VectorSubcoreMesh(core_axis_name="c", subcore_axis_name="s")
    return pl.kernel(
        _sc_body,
        out_type=jax.ShapeDtypeStruct((_F, 2, _NW, 8, 128), jnp.float32),
        mesh=mesh,
        scratch_types=[
            pltpu.VMEM((_CHUNK,), jnp.int32),
            [pltpu.VMEM((_H,), jnp.int32) for _ in range(_NPH)],
            [pltpu.VMEM((_H, _D), jnp.float32) for _ in range(_NPH)],
            pltpu.VMEM((_F, 2, 1, 8, 129), jnp.float32),
            [pltpu.SemaphoreType.DMA for _ in range(_NPH)],
        ],
        compiler_params=pltpu.CompilerParams(
            use_tc_tiling_on_sc=False, needs_layout_passes=False),
    )(x_flat, w_perm)


@jax.jit
def _run(x, weight, mask):
    w_perm = _permute_tc(weight.T, mask.T)
    x_flat = x.reshape(-1).astype(jnp.int32)
    out5 = _sc_lookup(x_flat, w_perm)
    out = out5.transpose(2, 4, 0, 1, 3).reshape(_B, _F, _D)
    return out


def kernel(x, weight, mask):
    return _run(x, weight, mask)
